# baseline (speedup 1.0000x reference)
import functools

import jax
import jax.numpy as jnp
from jax import lax
from jax.experimental import pallas as pl
from jax.experimental.pallas import tpu as pltpu
from jax.experimental.pallas import tpu_sc as plsc

VOCAB = 1_000_000
EMBED_DIM = 64
NUM_CLASSES = 100
BATCH = 4096
SEQ_LEN = 200
HALF = SEQ_LEN // 2

NUM_CORES = 2
NUM_SUBCORES = 16
NUM_WORKERS = NUM_CORES * NUM_SUBCORES
B_PER_W = BATCH // NUM_WORKERS
LANES = 16
D_CHUNKS = EMBED_DIM // LANES


_mesh = plsc.VectorSubcoreMesh(core_axis_name="c", subcore_axis_name="s")


@functools.partial(
    pl.kernel,
    mesh=_mesh,
    compiler_params=pltpu.CompilerParams(use_tc_tiling_on_sc=False),
    out_type=jax.ShapeDtypeStruct((BATCH, EMBED_DIM), jnp.float32),
    scratch_types=[
        pltpu.VMEM((B_PER_W, 2, HALF), jnp.int32),
        pltpu.VMEM((2, SEQ_LEN, EMBED_DIM), jnp.float32),
        pltpu.VMEM((B_PER_W, EMBED_DIM), jnp.float32),
        pltpu.SemaphoreType.DMA,
        pltpu.SemaphoreType.DMA,
    ],
)
def _pool(tok_hbm, table_hbm, out_hbm, idx_v, rows_v, out_v, sem0, sem1):
    wid = lax.axis_index("s") * NUM_CORES + lax.axis_index("c")
    base = wid * B_PER_W
    sems = (sem0, sem1)

    pltpu.sync_copy(tok_hbm.at[pl.ds(base, B_PER_W)], idx_v)

    def issue(u, buf):
        pltpu.async_copy(
            table_hbm.at[idx_v.at[u, 0]],
            rows_v.at[buf, pl.ds(0, HALF)],
            sems[buf],
        )
        pltpu.async_copy(
            table_hbm.at[idx_v.at[u, 1]],
            rows_v.at[buf, pl.ds(HALF, HALF)],
            sems[buf],
        )

    def drain(u, buf):
        pltpu.make_async_copy(
            table_hbm.at[idx_v.at[u, 0]],
            rows_v.at[buf, pl.ds(0, HALF)],
            sems[buf],
        ).wait()
        pltpu.make_async_copy(
            table_hbm.at[idx_v.at[u, 1]],
            rows_v.at[buf, pl.ds(HALF, HALF)],
            sems[buf],
        ).wait()

    inv = jnp.float32(1.0 / SEQ_LEN)
    zero = jnp.zeros((LANES,), jnp.float32)

    issue(0, 0)
    issue(1, 1)

    def outer(u0):
        for buf in range(2):
            u = u0 + buf
            drain(u, buf)
            rows_b = rows_v.at[buf]

            def acc_body(r, carry):
                return tuple(
                    carry[k] + rows_b[r, pl.ds(k * LANES, LANES)]
                    for k in range(D_CHUNKS)
                )

            acc = lax.fori_loop(0, SEQ_LEN, acc_body, (zero,) * D_CHUNKS)
            for k in range(D_CHUNKS):
                out_v[u, pl.ds(k * LANES, LANES)] = acc[k] * inv

            @pl.when(u + 2 < B_PER_W)
            def _():
                issue(u + 2, buf)

    lax.fori_loop(0, B_PER_W // 2, lambda i, _: (outer(2 * i), 0)[1], 0)

    pltpu.sync_copy(out_v, out_hbm.at[pl.ds(base, B_PER_W)])


def _head_body(p_ref, w_ref, b_ref, o_ref):
    o_ref[...] = (
        jnp.dot(p_ref[...], w_ref[...], preferred_element_type=jnp.float32)
        + b_ref[...]
    )


def _head(pooled, W, b):
    blk = 512
    return pl.pallas_call(
        _head_body,
        out_shape=jax.ShapeDtypeStruct((BATCH, NUM_CLASSES), jnp.float32),
        grid=(BATCH // blk,),
        in_specs=[
            pl.BlockSpec((blk, EMBED_DIM), lambda i: (i, 0)),
            pl.BlockSpec((EMBED_DIM, NUM_CLASSES), lambda i: (0, 0)),
            pl.BlockSpec((1, NUM_CLASSES), lambda i: (0, 0)),
        ],
        out_specs=pl.BlockSpec((blk, NUM_CLASSES), lambda i: (i, 0)),
    )(pooled, W, b.reshape(1, NUM_CLASSES))


def kernel(utteranceTokens, embedding_table, W, b):
    tok3 = utteranceTokens.astype(jnp.int32).reshape(BATCH, 2, HALF)
    pooled = _pool(tok3, embedding_table)
    return _head(pooled, W, b)


# speedup vs baseline: 1.0060x; 1.0060x over previous
import functools

import jax
import jax.numpy as jnp
from jax import lax
from jax.experimental import pallas as pl
from jax.experimental.pallas import tpu as pltpu
from jax.experimental.pallas import tpu_sc as plsc

VOCAB = 1_000_000
EMBED_DIM = 64
NUM_CLASSES = 100
BATCH = 4096
SEQ_LEN = 200
SLICE_A = 104
SLICE_B = 96

NUM_CORES = 2
NUM_SUBCORES = 16
NUM_WORKERS = NUM_CORES * NUM_SUBCORES
B_PER_W = BATCH // NUM_WORKERS
LANES = 16
D_CHUNKS = EMBED_DIM // LANES


_mesh = plsc.VectorSubcoreMesh(core_axis_name="c", subcore_axis_name="s")


@functools.partial(
    pl.kernel,
    mesh=_mesh,
    compiler_params=pltpu.CompilerParams(use_tc_tiling_on_sc=False),
    out_type=jax.ShapeDtypeStruct((BATCH, EMBED_DIM), jnp.float32),
    scratch_types=[
        pltpu.VMEM((B_PER_W, SEQ_LEN), jnp.int32),
        pltpu.VMEM((2, SEQ_LEN, EMBED_DIM), jnp.float32),
        pltpu.VMEM((B_PER_W, EMBED_DIM), jnp.float32),
        pltpu.SemaphoreType.DMA,
        pltpu.SemaphoreType.DMA,
    ],
)
def _pool(tok_hbm, table_hbm, out_hbm, idx_v, rows_v, out_v, sem0, sem1):
    wid = lax.axis_index("s") * NUM_CORES + lax.axis_index("c")
    base = wid * B_PER_W
    sems = (sem0, sem1)

    pltpu.sync_copy(tok_hbm.at[pl.ds(base, B_PER_W)], idx_v)

    def issue(u, buf):
        pltpu.async_copy(
            table_hbm.at[idx_v.at[u, pl.ds(0, SLICE_A)]],
            rows_v.at[buf, pl.ds(0, SLICE_A)],
            sems[buf],
        )
        pltpu.async_copy(
            table_hbm.at[idx_v.at[u, pl.ds(SLICE_A, SLICE_B)]],
            rows_v.at[buf, pl.ds(SLICE_A, SLICE_B)],
            sems[buf],
        )

    def drain(u, buf):
        pltpu.make_async_copy(
            table_hbm.at[idx_v.at[u, pl.ds(0, SLICE_A)]],
            rows_v.at[buf, pl.ds(0, SLICE_A)],
            sems[buf],
        ).wait()
        pltpu.make_async_copy(
            table_hbm.at[idx_v.at[u, pl.ds(SLICE_A, SLICE_B)]],
            rows_v.at[buf, pl.ds(SLICE_A, SLICE_B)],
            sems[buf],
        ).wait()

    inv = jnp.float32(1.0 / SEQ_LEN)
    zero = jnp.zeros((LANES,), jnp.float32)

    issue(0, 0)
    issue(1, 1)

    def outer(u0):
        for buf in range(2):
            u = u0 + buf
            drain(u, buf)
            rows_b = rows_v.at[buf]

            def acc_body(r, carry):
                return tuple(
                    carry[k] + rows_b[r, pl.ds(k * LANES, LANES)]
                    for k in range(D_CHUNKS)
                )

            acc = lax.fori_loop(0, SEQ_LEN, acc_body, (zero,) * D_CHUNKS)
            for k in range(D_CHUNKS):
                out_v[u, pl.ds(k * LANES, LANES)] = acc[k] * inv

            @pl.when(u + 2 < B_PER_W)
            def _():
                issue(u + 2, buf)

    lax.fori_loop(0, B_PER_W // 2, lambda i, _: (outer(2 * i), 0)[1], 0)

    pltpu.sync_copy(out_v, out_hbm.at[pl.ds(base, B_PER_W)])


def _head_body(p_ref, w_ref, b_ref, o_ref):
    o_ref[...] = (
        jnp.dot(p_ref[...], w_ref[...], preferred_element_type=jnp.float32)
        + b_ref[...]
    )


def _head(pooled, W, b):
    blk = 512
    return pl.pallas_call(
        _head_body,
        out_shape=jax.ShapeDtypeStruct((BATCH, NUM_CLASSES), jnp.float32),
        grid=(BATCH // blk,),
        in_specs=[
            pl.BlockSpec((blk, EMBED_DIM), lambda i: (i, 0)),
            pl.BlockSpec((EMBED_DIM, NUM_CLASSES), lambda i: (0, 0)),
            pl.BlockSpec((1, NUM_CLASSES), lambda i: (0, 0)),
        ],
        out_specs=pl.BlockSpec((blk, NUM_CLASSES), lambda i: (i, 0)),
    )(pooled, W, b.reshape(1, NUM_CLASSES))


def kernel(utteranceTokens, embedding_table, W, b):
    tok = utteranceTokens.astype(jnp.int32)
    pooled = _pool(tok, embedding_table)
    return _head(pooled, W, b)
